# Initial kernel scaffold; baseline (speedup 1.0000x reference)
import jax
import jax.numpy as jnp
from jax.experimental import pallas as pl


def kernel(x, edge_index, edge_attr, batch, atom_emb, bond_emb, W1_0, b1_0, W2_0, b2_0, W1_1, b1_1, W2_1, b2_1, W1_2, b1_2, W2_2, b2_2):
    raise NotImplementedError("write your pallas kernel here")


# speedup vs baseline: 3.6099x; 3.6099x over previous
import functools

import jax
import jax.numpy as jnp
from jax import lax
from jax.experimental import pallas as pl
from jax.experimental.pallas import tpu as pltpu
from jax.experimental.pallas import tpu_sc as plsc

N = 50000
E = 800000
D = 64
G = 128
HALF = 32
NSUB = 16

NP = 50176
NROWS_T = NP // NSUB
ZROWS = 196

EP = 16 * 50048
ET_E = EP // NSUB
CE = 128
NCH_E = ET_E // CE

ATAB = 1280
CTAB = 512
AT = 9 * NP
ET_A = AT // NSUB
CA = 112
NCH_A = ET_A // CA

BM = 1024
NB = NP // BM


@functools.cache
def _mesh():
    return plsc.VectorSubcoreMesh(core_axis_name="c", subcore_axis_name="s",
                                  num_cores=2, num_subcores=NSUB)


def _zero_acc(zbuf_v, acc, s):
    zero16 = jnp.zeros((16,), jnp.float32)

    def zb(i, carry):
        zbuf_v[i, pl.ds(0, 16)] = zero16
        zbuf_v[i, pl.ds(16, 16)] = zero16
        return carry

    lax.fori_loop(0, ZROWS, zb, 0)

    def zc(k, carry):
        pltpu.sync_copy(zbuf_v, acc.at[pl.ds(s * NROWS_T + k * ZROWS, ZROWS)])
        return carry

    lax.fori_loop(0, NROWS_T // ZROWS, zc, 0)


def _sc_atom_body(xidx_hbm, nid_hbm, atab_hbm, out_hbm,
                  idx_v, dst_v, rows_v, zbuf_v, acc, sem):
    c = lax.axis_index("c")
    s = lax.axis_index("s")
    _zero_acc(zbuf_v, acc, s)
    plsc.subcore_barrier()
    toff = c * ATAB

    def chunk(k, carry):
        base = s * ET_A + k * CA
        pltpu.sync_copy(xidx_hbm.at[pl.ds(base, CA)], idx_v)
        pltpu.sync_copy(nid_hbm.at[pl.ds(base, CA)], dst_v)

        def off(j, ocarry):
            idx_v[pl.ds(j * 16, 16)] = idx_v[pl.ds(j * 16, 16)] + toff
            return ocarry

        lax.fori_loop(0, CA // 16, off, 0)
        pltpu.async_copy(atab_hbm.at[idx_v], rows_v, sem).wait()
        pltpu.sync_copy(rows_v, acc.at[dst_v], add=True)
        return carry

    lax.fori_loop(0, NCH_A, chunk, 0)
    plsc.subcore_barrier()
    pltpu.sync_copy(acc.at[pl.ds(s * NROWS_T, NROWS_T)],
                    out_hbm.at[pl.ds(c * NP + s * NROWS_T, NROWS_T)])


def _sc_layer_body(src_hbm, dst_hbm, code_hbm, h_hbm, comb_hbm, out_hbm,
                   sidx_v, didx_v, cidx_v, hrow_v, erow_v, zbuf_v, acc,
                   sem1, sem2):
    c = lax.axis_index("c")
    s = lax.axis_index("s")
    _zero_acc(zbuf_v, acc, s)
    plsc.subcore_barrier()
    hoff = c * NP
    coff = c * CTAB

    def chunk(k, carry):
        base = s * ET_E + k * CE
        pltpu.sync_copy(src_hbm.at[pl.ds(base, CE)], sidx_v)
        pltpu.sync_copy(code_hbm.at[pl.ds(base, CE)], cidx_v)
        pltpu.sync_copy(dst_hbm.at[pl.ds(base, CE)], didx_v)

        def off(j, ocarry):
            sidx_v[pl.ds(j * 16, 16)] = sidx_v[pl.ds(j * 16, 16)] + hoff
            cidx_v[pl.ds(j * 16, 16)] = cidx_v[pl.ds(j * 16, 16)] + coff
            return ocarry

        lax.fori_loop(0, CE // 16, off, 0)
        g1 = pltpu.async_copy(h_hbm.at[sidx_v], hrow_v, sem1)
        g2 = pltpu.async_copy(comb_hbm.at[cidx_v], erow_v, sem2)
        g1.wait()
        g2.wait()

        def vec(r, vcarry):
            a = hrow_v[r, pl.ds(0, 16)] + erow_v[r, pl.ds(0, 16)]
            hrow_v[r, pl.ds(0, 16)] = jnp.maximum(a, 0.0)
            b = hrow_v[r, pl.ds(16, 16)] + erow_v[r, pl.ds(16, 16)]
            hrow_v[r, pl.ds(16, 16)] = jnp.maximum(b, 0.0)
            return vcarry

        lax.fori_loop(0, CE, vec, 0)
        pltpu.sync_copy(hrow_v, acc.at[didx_v], add=True)
        return carry

    lax.fori_loop(0, NCH_E, chunk, 0)
    plsc.subcore_barrier()
    pltpu.sync_copy(acc.at[pl.ds(s * NROWS_T, NROWS_T)],
                    out_hbm.at[pl.ds(c * NP + s * NROWS_T, NROWS_T)])


@functools.cache
def _sc_atom():
    return pl.kernel(
        _sc_atom_body,
        out_type=jax.ShapeDtypeStruct((2 * NP, HALF), jnp.float32),
        mesh=_mesh(),
        compiler_params=pltpu.CompilerParams(use_tc_tiling_on_sc=False),
        scratch_types=[
            pltpu.VMEM((CA,), jnp.int32),
            pltpu.VMEM((CA,), jnp.int32),
            pltpu.VMEM((CA, HALF), jnp.float32),
            pltpu.VMEM((ZROWS, HALF), jnp.float32),
            pltpu.VMEM_SHARED((NP, HALF), jnp.float32),
            pltpu.SemaphoreType.DMA,
        ],
    )


@functools.cache
def _sc_layer():
    return pl.kernel(
        _sc_layer_body,
        out_type=jax.ShapeDtypeStruct((2 * NP, HALF), jnp.float32),
        mesh=_mesh(),
        compiler_params=pltpu.CompilerParams(use_tc_tiling_on_sc=False),
        scratch_types=[
            pltpu.VMEM((CE,), jnp.int32),
            pltpu.VMEM((CE,), jnp.int32),
            pltpu.VMEM((CE,), jnp.int32),
            pltpu.VMEM((CE, HALF), jnp.float32),
            pltpu.VMEM((CE, HALF), jnp.float32),
            pltpu.VMEM((ZROWS, HALF), jnp.float32),
            pltpu.VMEM_SHARED((NP, HALF), jnp.float32),
            pltpu.SemaphoreType.DMA,
            pltpu.SemaphoreType.DMA,
        ],
    )


def _mlp_compute(hlo, hhi, alo, ahi, w1, b1, w2, b2, last):
    h = jnp.concatenate([hlo[...], hhi[...]], axis=1)
    a = jnp.concatenate([alo[...], ahi[...]], axis=1)
    z = h + a
    z1 = jnp.maximum(
        jnp.dot(z, w1[...], preferred_element_type=jnp.float32) + b1[...], 0.0)
    h2 = jnp.dot(z1, w2[...], preferred_element_type=jnp.float32) + b2[...]
    if not last:
        h2 = jnp.maximum(h2, 0.0)
    return h2


def _mlp_body(hlo, hhi, alo, ahi, w1, b1, w2, b2, olo, ohi):
    h2 = _mlp_compute(hlo, hhi, alo, ahi, w1, b1, w2, b2, last=False)
    olo[...] = h2[:, :HALF]
    ohi[...] = h2[:, HALF:]


def _final_body(hlo, hhi, alo, ahi, w1, b1, w2, b2, bat, out):
    i = pl.program_id(0)
    h2 = _mlp_compute(hlo, hhi, alo, ahi, w1, b1, w2, b2, last=True)
    bids = bat[0]
    gids = lax.broadcasted_iota(jnp.int32, (G, BM), 0)
    rows = lax.broadcasted_iota(jnp.int32, (G, BM), 1) + i * BM
    onehot = jnp.where((bids == gids) & (rows < N), 1.0, 0.0)
    part = jnp.dot(onehot, h2, preferred_element_type=jnp.float32)

    @pl.when(i == 0)
    def _():
        out[...] = part

    @pl.when(i > 0)
    def _():
        out[...] = out[...] + part


_lo_spec = pl.BlockSpec((BM, HALF), lambda i: (i, 0))
_hi_spec = pl.BlockSpec((BM, HALF), lambda i: (NB + i, 0))
_w_spec = pl.BlockSpec((D, D), lambda i: (0, 0))
_b_spec = pl.BlockSpec((1, D), lambda i: (0, 0))

_mlp_call = pl.pallas_call(
    _mlp_body,
    grid=(NB,),
    in_specs=[_lo_spec, _hi_spec, _lo_spec, _hi_spec,
              _w_spec, _b_spec, _w_spec, _b_spec],
    out_specs=[pl.BlockSpec((BM, HALF), lambda i: (i, 0)),
               pl.BlockSpec((BM, HALF), lambda i: (i, 0))],
    out_shape=(jax.ShapeDtypeStruct((NP, HALF), jnp.float32),
               jax.ShapeDtypeStruct((NP, HALF), jnp.float32)),
)

_final_call = pl.pallas_call(
    _final_body,
    grid=(NB,),
    in_specs=[_lo_spec, _hi_spec, _lo_spec, _hi_spec,
              _w_spec, _b_spec, _w_spec, _b_spec,
              pl.BlockSpec((1, 1, BM), lambda i: (i, 0, 0))],
    out_specs=pl.BlockSpec((G, D), lambda i: (0, 0)),
    out_shape=jax.ShapeDtypeStruct((G, D), jnp.float32),
)


def kernel(x, edge_index, edge_attr, batch, atom_emb, bond_emb,
           W1_0, b1_0, W2_0, b2_0,
           W1_1, b1_1, W2_1, b2_1,
           W1_2, b1_2, W2_2, b2_2):
    atab = jnp.zeros((ATAB, D), jnp.float32).at[:1152].set(
        atom_emb.reshape(9 * 128, D))
    atab_stack = jnp.concatenate([atab[:, :HALF], atab[:, HALF:]], axis=0)
    comb = (bond_emb[0][:, None, None, :] + bond_emb[1][None, :, None, :]
            + bond_emb[2][None, None, :, :]).reshape(CTAB, D)
    comb_stack = jnp.concatenate([comb[:, :HALF], comb[:, HALF:]], axis=0)

    xi = x.astype(jnp.int32) + (128 * jnp.arange(9, dtype=jnp.int32))[None, :]
    xflat = jnp.pad(xi.T, ((0, 0), (0, NP - N)),
                    constant_values=1152).reshape(-1)
    nid = jnp.broadcast_to(jnp.arange(NP, dtype=jnp.int32)[None, :],
                           (9, NP)).reshape(-1)
    src = jnp.pad(edge_index[0], (0, EP - E))
    dst = jnp.pad(edge_index[1], (0, EP - E), constant_values=N)
    code = edge_attr[:, 0] * 64 + edge_attr[:, 1] * 8 + edge_attr[:, 2]
    code = jnp.pad(code, (0, EP - E))
    batp = jnp.pad(batch, (0, NP - N)).reshape(NB, 1, BM)
    b1s = (b1_0.reshape(1, D), b1_1.reshape(1, D), b1_2.reshape(1, D))
    b2s = (b2_0.reshape(1, D), b2_1.reshape(1, D), b2_2.reshape(1, D))
    w1s = (W1_0, W1_1, W1_2)
    w2s = (W2_0, W2_1, W2_2)

    h = _sc_atom()(xflat, nid, atab_stack)
    for l in range(2):
        agg = _sc_layer()(src, dst, code, h, comb_stack)
        olo, ohi = _mlp_call(h, h, agg, agg,
                             w1s[l], b1s[l], w2s[l], b2s[l])
        h = jnp.concatenate([olo, ohi], axis=0)
    agg = _sc_layer()(src, dst, code, h, comb_stack)
    return _final_call(h, h, agg, agg,
                       w1s[2], b1s[2], w2s[2], b2s[2], batp)


# speedup vs baseline: 8.5445x; 2.3670x over previous
import functools

import jax
import jax.numpy as jnp
from jax import lax
from jax.experimental import pallas as pl
from jax.experimental.pallas import tpu as pltpu
from jax.experimental.pallas import tpu_sc as plsc

N = 50000
E = 800000
D = 64
G = 128
HALF = 32
NSUB = 16

NP = 50176
NROWS_T = NP // NSUB
ZROWS = 98

CE = 128
KI = 8
EP = 16 * 50176
NCHT = 392
NBLK_E = NCHT // KI

ATAB = 1152
NA = 14
CA = 9 * NA
NCHA = NROWS_T // NA
NBLK_A = NCHA // KI
CTAB = 512

BM = 1024
NB = NP // BM


@functools.cache
def _mesh():
    return plsc.VectorSubcoreMesh(core_axis_name="c", subcore_axis_name="s",
                                  num_cores=2, num_subcores=NSUB)


def _zero_acc(zbuf_v, acc, s):
    zero16 = jnp.zeros((16,), jnp.float32)

    def zb(i, carry):
        zbuf_v[i, pl.ds(0, 16)] = zero16
        zbuf_v[i, pl.ds(16, 16)] = zero16
        return carry

    lax.fori_loop(0, ZROWS, zb, 0)

    def zc(k, carry):
        pltpu.sync_copy(zbuf_v, acc.at[pl.ds(s * NROWS_T + k * ZROWS, ZROWS)])
        return carry

    lax.fori_loop(0, NROWS_T // ZROWS, zc, 0)


def _sc_atom_body(xidx_hbm, atab_hbm, out_hbm,
                  idx_v, rows_v, nacc_v, atab_spm, semi, semg):
    c = lax.axis_index("c")
    s = lax.axis_index("s")
    pltpu.sync_copy(atab_hbm.at[pl.ds(c * ATAB + s * (ATAB // NSUB),
                                      ATAB // NSUB)],
                    atab_spm.at[pl.ds(s * (ATAB // NSUB), ATAB // NSUB)])
    plsc.subcore_barrier()

    ibase = s * NCHA

    def block(b, carry):
        pltpu.async_copy(xidx_hbm.at[pl.ds(ibase + b * KI, KI)],
                         idx_v, semi).wait()
        for j in range(KI):
            pltpu.async_copy(atab_spm.at[idx_v.at[j]], rows_v, semg).wait()

            def node(m, ncarry):
                t = m * 9
                for lo in (0, 16):
                    r0 = rows_v[t, pl.ds(lo, 16)] + rows_v[t + 1, pl.ds(lo, 16)]
                    r1 = rows_v[t + 2, pl.ds(lo, 16)] + rows_v[t + 3, pl.ds(lo, 16)]
                    r2 = rows_v[t + 4, pl.ds(lo, 16)] + rows_v[t + 5, pl.ds(lo, 16)]
                    r3 = rows_v[t + 6, pl.ds(lo, 16)] + rows_v[t + 7, pl.ds(lo, 16)]
                    r4 = (r0 + r1) + (r2 + r3)
                    nacc_v[j * NA + m, pl.ds(lo, 16)] = (
                        r4 + rows_v[t + 8, pl.ds(lo, 16)])
                return ncarry

            lax.fori_loop(0, NA, node, 0)
        pltpu.sync_copy(
            nacc_v,
            out_hbm.at[pl.ds(c * NP + s * NROWS_T + b * (KI * NA), KI * NA)])
        return carry

    lax.fori_loop(0, NBLK_A, block, 0)


def _sc_layer_body(srcS_hbm, dstS_hbm, codeS_hbm, h_hbm, comb_hbm, out_hbm,
                   sidx_v, didx_v, cidx_v, hrow_v, erow_v, zbuf_v,
                   comb_spm, acc, semi, semh0, semh1):
    c = lax.axis_index("c")
    s = lax.axis_index("s")
    pltpu.sync_copy(comb_hbm.at[pl.ds(c * CTAB + s * (CTAB // NSUB),
                                      CTAB // NSUB)],
                    comb_spm.at[pl.ds(s * (CTAB // NSUB), CTAB // NSUB)])
    _zero_acc(zbuf_v, acc, s)
    plsc.subcore_barrier()

    sbase = c * (EP // CE) + s * NCHT
    obase = s * NCHT

    def block(b, carry):
        r1 = pltpu.async_copy(srcS_hbm.at[pl.ds(sbase + b * KI, KI)],
                              sidx_v, semi)
        r2 = pltpu.async_copy(codeS_hbm.at[pl.ds(obase + b * KI, KI)],
                              cidx_v, semi)
        r3 = pltpu.async_copy(dstS_hbm.at[pl.ds(obase + b * KI, KI)],
                              didx_v, semi)
        r1.wait()
        r2.wait()
        r3.wait()
        pending = pltpu.async_copy(h_hbm.at[sidx_v.at[0]], hrow_v.at[0], semh0)
        for j in range(KI):
            p = j % 2
            if j < KI - 1:
                nxt = pltpu.async_copy(h_hbm.at[sidx_v.at[j + 1]],
                                       hrow_v.at[1 - p],
                                       semh1 if p == 0 else semh0)
            pending.wait()
            if j < KI - 1:
                pending = nxt
            pltpu.sync_copy(comb_spm.at[cidx_v.at[j]], erow_v)
            hr = hrow_v.at[p]

            def vec(r, vcarry):
                i0 = r * 2
                for dd in (0, 1):
                    for lo in (0, 16):
                        a = (hr[i0 + dd, pl.ds(lo, 16)]
                             + erow_v[i0 + dd, pl.ds(lo, 16)])
                        hr[i0 + dd, pl.ds(lo, 16)] = jnp.maximum(a, 0.0)
                return vcarry

            lax.fori_loop(0, CE // 2, vec, 0)
            pltpu.sync_copy(hr, acc.at[didx_v.at[j]], add=True)
        return carry

    lax.fori_loop(0, NBLK_E, block, 0)
    plsc.subcore_barrier()
    pltpu.sync_copy(acc.at[pl.ds(s * NROWS_T, NROWS_T)],
                    out_hbm.at[pl.ds(c * NP + s * NROWS_T, NROWS_T)])


@functools.cache
def _sc_atom():
    return pl.kernel(
        _sc_atom_body,
        out_type=jax.ShapeDtypeStruct((2 * NP, HALF), jnp.float32),
        mesh=_mesh(),
        compiler_params=pltpu.CompilerParams(use_tc_tiling_on_sc=False),
        scratch_types=[
            pltpu.VMEM((KI, CA), jnp.int32),
            pltpu.VMEM((CA, HALF), jnp.float32),
            pltpu.VMEM((KI * NA, HALF), jnp.float32),
            pltpu.VMEM_SHARED((ATAB, HALF), jnp.float32),
            pltpu.SemaphoreType.DMA,
            pltpu.SemaphoreType.DMA,
        ],
    )


@functools.cache
def _sc_layer():
    return pl.kernel(
        _sc_layer_body,
        out_type=jax.ShapeDtypeStruct((2 * NP, HALF), jnp.float32),
        mesh=_mesh(),
        compiler_params=pltpu.CompilerParams(use_tc_tiling_on_sc=False),
        scratch_types=[
            pltpu.VMEM((KI, CE), jnp.int32),
            pltpu.VMEM((KI, CE), jnp.int32),
            pltpu.VMEM((KI, CE), jnp.int32),
            pltpu.VMEM((2, CE, HALF), jnp.float32),
            pltpu.VMEM((CE, HALF), jnp.float32),
            pltpu.VMEM((ZROWS, HALF), jnp.float32),
            pltpu.VMEM_SHARED((CTAB, HALF), jnp.float32),
            pltpu.VMEM_SHARED((NP, HALF), jnp.float32),
            pltpu.SemaphoreType.DMA,
            pltpu.SemaphoreType.DMA,
            pltpu.SemaphoreType.DMA,
        ],
    )


def _mlp_compute(hlo, hhi, alo, ahi, w1, b1, w2, b2, last):
    h = jnp.concatenate([hlo[...], hhi[...]], axis=1)
    a = jnp.concatenate([alo[...], ahi[...]], axis=1)
    z = h + a
    z1 = jnp.maximum(
        jnp.dot(z, w1[...], preferred_element_type=jnp.float32) + b1[...], 0.0)
    h2 = jnp.dot(z1, w2[...], preferred_element_type=jnp.float32) + b2[...]
    if not last:
        h2 = jnp.maximum(h2, 0.0)
    return h2


def _mlp_body(hlo, hhi, alo, ahi, w1, b1, w2, b2, olo, ohi):
    h2 = _mlp_compute(hlo, hhi, alo, ahi, w1, b1, w2, b2, last=False)
    olo[...] = h2[:, :HALF]
    ohi[...] = h2[:, HALF:]


def _final_body(hlo, hhi, alo, ahi, w1, b1, w2, b2, bat, out):
    i = pl.program_id(0)
    h2 = _mlp_compute(hlo, hhi, alo, ahi, w1, b1, w2, b2, last=True)
    bids = bat[0]
    gids = lax.broadcasted_iota(jnp.int32, (G, BM), 0)
    rows = lax.broadcasted_iota(jnp.int32, (G, BM), 1) + i * BM
    onehot = jnp.where((bids == gids) & (rows < N), 1.0, 0.0)
    part = jnp.dot(onehot, h2, preferred_element_type=jnp.float32)

    @pl.when(i == 0)
    def _():
        out[...] = part

    @pl.when(i > 0)
    def _():
        out[...] = out[...] + part


_lo_spec = pl.BlockSpec((BM, HALF), lambda i: (i, 0))
_hi_spec = pl.BlockSpec((BM, HALF), lambda i: (NB + i, 0))
_w_spec = pl.BlockSpec((D, D), lambda i: (0, 0))
_b_spec = pl.BlockSpec((1, D), lambda i: (0, 0))

_mlp_call = pl.pallas_call(
    _mlp_body,
    grid=(NB,),
    in_specs=[_lo_spec, _hi_spec, _lo_spec, _hi_spec,
              _w_spec, _b_spec, _w_spec, _b_spec],
    out_specs=[pl.BlockSpec((BM, HALF), lambda i: (i, 0)),
               pl.BlockSpec((BM, HALF), lambda i: (i, 0))],
    out_shape=(jax.ShapeDtypeStruct((NP, HALF), jnp.float32),
               jax.ShapeDtypeStruct((NP, HALF), jnp.float32)),
)

_final_call = pl.pallas_call(
    _final_body,
    grid=(NB,),
    in_specs=[_lo_spec, _hi_spec, _lo_spec, _hi_spec,
              _w_spec, _b_spec, _w_spec, _b_spec,
              pl.BlockSpec((1, 1, BM), lambda i: (i, 0, 0))],
    out_specs=pl.BlockSpec((G, D), lambda i: (0, 0)),
    out_shape=jax.ShapeDtypeStruct((G, D), jnp.float32),
)


def kernel(x, edge_index, edge_attr, batch, atom_emb, bond_emb,
           W1_0, b1_0, W2_0, b2_0,
           W1_1, b1_1, W2_1, b2_1,
           W1_2, b1_2, W2_2, b2_2):
    atab = atom_emb.reshape(ATAB, D)
    atab_stack = jnp.concatenate([atab[:, :HALF], atab[:, HALF:]], axis=0)
    comb = (bond_emb[0][:, None, None, :] + bond_emb[1][None, :, None, :]
            + bond_emb[2][None, None, :, :]).reshape(CTAB, D)
    comb_stack = jnp.concatenate([comb[:, :HALF], comb[:, HALF:]], axis=0)

    pad_n = NP - N
    xi = x.astype(jnp.int32) + (128 * jnp.arange(9, dtype=jnp.int32))[None, :]
    xpad = (jnp.arange(pad_n * 9, dtype=jnp.int32) * 37) % ATAB
    xflat = jnp.concatenate([xi.reshape(-1), xpad]).reshape(NP * 9 // CA, CA)

    pad_e = EP - E
    ar = jnp.arange(pad_e, dtype=jnp.int32)
    src = jnp.concatenate([edge_index[0], N + ar % pad_n])
    srcS = jnp.concatenate([src, src + NP]).reshape(2 * EP // CE, CE)
    dstS = jnp.concatenate([edge_index[1], N + ar % pad_n]
                           ).reshape(EP // CE, CE)
    code = edge_attr[:, 0] * 64 + edge_attr[:, 1] * 8 + edge_attr[:, 2]
    codeS = jnp.concatenate([code, ar % CTAB]).reshape(EP // CE, CE)

    batp = jnp.pad(batch, (0, NP - N)).reshape(NB, 1, BM)
    b1s = (b1_0.reshape(1, D), b1_1.reshape(1, D), b1_2.reshape(1, D))
    b2s = (b2_0.reshape(1, D), b2_1.reshape(1, D), b2_2.reshape(1, D))
    w1s = (W1_0, W1_1, W1_2)
    w2s = (W2_0, W2_1, W2_2)

    h = _sc_atom()(xflat, atab_stack)
    for l in range(2):
        agg = _sc_layer()(srcS, dstS, codeS, h, comb_stack)
        olo, ohi = _mlp_call(h, h, agg, agg,
                             w1s[l], b1s[l], w2s[l], b2s[l])
        h = jnp.concatenate([olo, ohi], axis=0)
    agg = _sc_layer()(srcS, dstS, codeS, h, comb_stack)
    return _final_call(h, h, agg, agg,
                       w1s[2], b1s[2], w2s[2], b2s[2], batp)


# speedup vs baseline: 10.3668x; 1.2133x over previous
import functools

import jax
import jax.numpy as jnp
from jax import lax
from jax.experimental import pallas as pl
from jax.experimental.pallas import tpu as pltpu
from jax.experimental.pallas import tpu_sc as plsc

N = 50000
E = 800000
D = 64
G = 128
HALF = 32
NSUB = 16

NP = 50176
NROWS_T = NP // NSUB
ZROWS = 98

CE = 128
KI = 8
EP = 16 * 50176
NCHT = 392
NBLK_E = NCHT // KI

ATAB = 1152
NA = 14
CA = 9 * NA
NCHA = NROWS_T // NA
NBLK_A = NCHA // KI
CTAB = 512

BM = 1024
NB = NP // BM


@functools.cache
def _mesh():
    return plsc.VectorSubcoreMesh(core_axis_name="c", subcore_axis_name="s",
                                  num_cores=2, num_subcores=NSUB)


def _zero_acc(zbuf_v, acc, s):
    zero16 = jnp.zeros((16,), jnp.float32)

    def zb(i, carry):
        zbuf_v[i, pl.ds(0, 16)] = zero16
        zbuf_v[i, pl.ds(16, 16)] = zero16
        return carry

    lax.fori_loop(0, ZROWS, zb, 0)

    def zc(k, carry):
        pltpu.sync_copy(zbuf_v, acc.at[pl.ds(s * NROWS_T + k * ZROWS, ZROWS)])
        return carry

    lax.fori_loop(0, NROWS_T // ZROWS, zc, 0)


def _sc_atom_body(xidx_hbm, atab_hbm, out_hbm,
                  idx_v, rows_v, nacc_v, atab_spm, semi, semg):
    c = lax.axis_index("c")
    s = lax.axis_index("s")
    pltpu.sync_copy(atab_hbm.at[pl.ds(c * ATAB + s * (ATAB // NSUB),
                                      ATAB // NSUB)],
                    atab_spm.at[pl.ds(s * (ATAB // NSUB), ATAB // NSUB)])
    plsc.subcore_barrier()

    ibase = s * NCHA

    def block(b, carry):
        pltpu.async_copy(xidx_hbm.at[pl.ds(ibase + b * KI, KI)],
                         idx_v, semi).wait()
        for j in range(KI):
            pltpu.async_copy(atab_spm.at[idx_v.at[j]], rows_v, semg).wait()

            def node(m, ncarry):
                t = m * 9
                for lo in (0, 16):
                    r0 = rows_v[t, pl.ds(lo, 16)] + rows_v[t + 1, pl.ds(lo, 16)]
                    r1 = rows_v[t + 2, pl.ds(lo, 16)] + rows_v[t + 3, pl.ds(lo, 16)]
                    r2 = rows_v[t + 4, pl.ds(lo, 16)] + rows_v[t + 5, pl.ds(lo, 16)]
                    r3 = rows_v[t + 6, pl.ds(lo, 16)] + rows_v[t + 7, pl.ds(lo, 16)]
                    r4 = (r0 + r1) + (r2 + r3)
                    nacc_v[j * NA + m, pl.ds(lo, 16)] = (
                        r4 + rows_v[t + 8, pl.ds(lo, 16)])
                return ncarry

            lax.fori_loop(0, NA, node, 0)
        pltpu.sync_copy(
            nacc_v,
            out_hbm.at[pl.ds(c * NP + s * NROWS_T + b * (KI * NA), KI * NA)])
        return carry

    lax.fori_loop(0, NBLK_A, block, 0)


def _sc_layer_body(srcS_hbm, dstS_hbm, codeS_hbm, h_hbm, comb_hbm, out_hbm,
                   sidx_v, didx_v, cidx_v, hrow_v, erow_v, zbuf_v,
                   comb_spm, acc, semi, semh0, semh1, seme0, seme1):
    c = lax.axis_index("c")
    s = lax.axis_index("s")
    pltpu.sync_copy(comb_hbm.at[pl.ds(c * CTAB + s * (CTAB // NSUB),
                                      CTAB // NSUB)],
                    comb_spm.at[pl.ds(s * (CTAB // NSUB), CTAB // NSUB)])
    _zero_acc(zbuf_v, acc, s)
    plsc.subcore_barrier()

    sbase = c * (EP // CE) + s * NCHT
    obase = s * NCHT

    def block(b, carry):
        r1 = pltpu.async_copy(srcS_hbm.at[pl.ds(sbase + b * KI, KI)],
                              sidx_v, semi)
        r2 = pltpu.async_copy(codeS_hbm.at[pl.ds(obase + b * KI, KI)],
                              cidx_v, semi)
        r3 = pltpu.async_copy(dstS_hbm.at[pl.ds(obase + b * KI, KI)],
                              didx_v, semi)
        r1.wait()
        r2.wait()
        r3.wait()
        ph = pltpu.async_copy(h_hbm.at[sidx_v.at[0]], hrow_v.at[0], semh0)
        pe = pltpu.async_copy(comb_spm.at[cidx_v.at[0]], erow_v.at[0], seme0)
        for j in range(KI):
            p = j % 2
            if j < KI - 1:
                nh = pltpu.async_copy(h_hbm.at[sidx_v.at[j + 1]],
                                      hrow_v.at[1 - p],
                                      semh1 if p == 0 else semh0)
                ne = pltpu.async_copy(comb_spm.at[cidx_v.at[j + 1]],
                                      erow_v.at[1 - p],
                                      seme1 if p == 0 else seme0)
            ph.wait()
            pe.wait()
            if j < KI - 1:
                ph = nh
                pe = ne
            hr = hrow_v.at[p]
            er = erow_v.at[p]

            def vec(r, vcarry):
                i0 = r * 4
                for dd in (0, 1, 2, 3):
                    for lo in (0, 16):
                        a = (hr[i0 + dd, pl.ds(lo, 16)]
                             + er[i0 + dd, pl.ds(lo, 16)])
                        hr[i0 + dd, pl.ds(lo, 16)] = jnp.maximum(a, 0.0)
                return vcarry

            lax.fori_loop(0, CE // 4, vec, 0)
            pltpu.sync_copy(hr, acc.at[didx_v.at[j]], add=True)
        return carry

    lax.fori_loop(0, NBLK_E, block, 0)
    plsc.subcore_barrier()
    pltpu.sync_copy(acc.at[pl.ds(s * NROWS_T, NROWS_T)],
                    out_hbm.at[pl.ds(c * NP + s * NROWS_T, NROWS_T)])


@functools.cache
def _sc_atom():
    return pl.kernel(
        _sc_atom_body,
        out_type=jax.ShapeDtypeStruct((2 * NP, HALF), jnp.float32),
        mesh=_mesh(),
        compiler_params=pltpu.CompilerParams(use_tc_tiling_on_sc=False),
        scratch_types=[
            pltpu.VMEM((KI, CA), jnp.int32),
            pltpu.VMEM((CA, HALF), jnp.float32),
            pltpu.VMEM((KI * NA, HALF), jnp.float32),
            pltpu.VMEM_SHARED((ATAB, HALF), jnp.float32),
            pltpu.SemaphoreType.DMA,
            pltpu.SemaphoreType.DMA,
        ],
    )


@functools.cache
def _sc_layer():
    return pl.kernel(
        _sc_layer_body,
        out_type=jax.ShapeDtypeStruct((2 * NP, HALF), jnp.float32),
        mesh=_mesh(),
        compiler_params=pltpu.CompilerParams(use_tc_tiling_on_sc=False),
        scratch_types=[
            pltpu.VMEM((KI, CE), jnp.int32),
            pltpu.VMEM((KI, CE), jnp.int32),
            pltpu.VMEM((KI, CE), jnp.int32),
            pltpu.VMEM((2, CE, HALF), jnp.float32),
            pltpu.VMEM((2, CE, HALF), jnp.float32),
            pltpu.VMEM((ZROWS, HALF), jnp.float32),
            pltpu.VMEM_SHARED((CTAB, HALF), jnp.float32),
            pltpu.VMEM_SHARED((NP, HALF), jnp.float32),
            pltpu.SemaphoreType.DMA,
            pltpu.SemaphoreType.DMA,
            pltpu.SemaphoreType.DMA,
            pltpu.SemaphoreType.DMA,
            pltpu.SemaphoreType.DMA,
        ],
    )


def _mlp_compute(h3, a3, w1, b1, w2, b2, last):
    zl = h3[0] + a3[0]
    zh = h3[1] + a3[1]
    z1 = jnp.maximum(
        jnp.dot(zl, w1[:HALF], preferred_element_type=jnp.float32)
        + jnp.dot(zh, w1[HALF:], preferred_element_type=jnp.float32)
        + b1[...], 0.0)
    h2 = jnp.dot(z1, w2[...], preferred_element_type=jnp.float32) + b2[...]
    if not last:
        h2 = jnp.maximum(h2, 0.0)
    return h2


def _mlp_body(h3, a3, w1, b1, w2, b2, o3):
    h2 = _mlp_compute(h3, a3, w1, b1, w2, b2, last=False)
    o3[0] = h2[:, :HALF]
    o3[1] = h2[:, HALF:]


def _final_body(h3, a3, w1, b1, w2, b2, bat, out):
    i = pl.program_id(0)
    h2 = _mlp_compute(h3, a3, w1, b1, w2, b2, last=True)
    bids = bat[0]
    gids = lax.broadcasted_iota(jnp.int32, (G, BM), 0)
    rows = lax.broadcasted_iota(jnp.int32, (G, BM), 1) + i * BM
    onehot = jnp.where((bids == gids) & (rows < N), 1.0, 0.0)
    part = jnp.dot(onehot, h2, preferred_element_type=jnp.float32)

    @pl.when(i == 0)
    def _():
        out[...] = part

    @pl.when(i > 0)
    def _():
        out[...] = out[...] + part


_h3_spec = pl.BlockSpec((2, BM, HALF), lambda i: (0, i, 0))
_w_spec = pl.BlockSpec((D, D), lambda i: (0, 0))
_b_spec = pl.BlockSpec((1, D), lambda i: (0, 0))

_mlp_call = pl.pallas_call(
    _mlp_body,
    grid=(NB,),
    in_specs=[_h3_spec, _h3_spec, _w_spec, _b_spec, _w_spec, _b_spec],
    out_specs=pl.BlockSpec((2, BM, HALF), lambda i: (0, i, 0)),
    out_shape=jax.ShapeDtypeStruct((2, NP, HALF), jnp.float32),
)

_final_call = pl.pallas_call(
    _final_body,
    grid=(NB,),
    in_specs=[_h3_spec, _h3_spec, _w_spec, _b_spec, _w_spec, _b_spec,
              pl.BlockSpec((1, 1, BM), lambda i: (i, 0, 0))],
    out_specs=pl.BlockSpec((G, D), lambda i: (0, 0)),
    out_shape=jax.ShapeDtypeStruct((G, D), jnp.float32),
)


def kernel(x, edge_index, edge_attr, batch, atom_emb, bond_emb,
           W1_0, b1_0, W2_0, b2_0,
           W1_1, b1_1, W2_1, b2_1,
           W1_2, b1_2, W2_2, b2_2):
    atab = atom_emb.reshape(ATAB, D)
    atab_stack = jnp.concatenate([atab[:, :HALF], atab[:, HALF:]], axis=0)
    comb = (bond_emb[0][:, None, None, :] + bond_emb[1][None, :, None, :]
            + bond_emb[2][None, None, :, :]).reshape(CTAB, D)
    comb_stack = jnp.concatenate([comb[:, :HALF], comb[:, HALF:]], axis=0)

    pad_n = NP - N
    xi = x.astype(jnp.int32) + (128 * jnp.arange(9, dtype=jnp.int32))[None, :]
    xpad = (jnp.arange(pad_n * 9, dtype=jnp.int32) * 37) % ATAB
    xflat = jnp.concatenate([xi.reshape(-1), xpad]).reshape(NP * 9 // CA, CA)

    pad_e = EP - E
    ar = jnp.arange(pad_e, dtype=jnp.int32)
    src = jnp.concatenate([edge_index[0], N + ar % pad_n])
    srcS = jnp.concatenate([src, src + NP]).reshape(2 * EP // CE, CE)
    dstS = jnp.concatenate([edge_index[1], N + ar % pad_n]
                           ).reshape(EP // CE, CE)
    code = edge_attr[:, 0] * 64 + edge_attr[:, 1] * 8 + edge_attr[:, 2]
    codeS = jnp.concatenate([code, ar % CTAB]).reshape(EP // CE, CE)

    batp = jnp.pad(batch, (0, NP - N)).reshape(NB, 1, BM)
    b1s = (b1_0.reshape(1, D), b1_1.reshape(1, D), b1_2.reshape(1, D))
    b2s = (b2_0.reshape(1, D), b2_1.reshape(1, D), b2_2.reshape(1, D))
    w1s = (W1_0, W1_1, W1_2)
    w2s = (W2_0, W2_1, W2_2)

    h = _sc_atom()(xflat, atab_stack)
    for l in range(2):
        agg = _sc_layer()(srcS, dstS, codeS, h, comb_stack)
        o3 = _mlp_call(h.reshape(2, NP, HALF), agg.reshape(2, NP, HALF),
                       w1s[l], b1s[l], w2s[l], b2s[l])
        h = o3.reshape(2 * NP, HALF)
    agg = _sc_layer()(srcS, dstS, codeS, h, comb_stack)
    return _final_call(h.reshape(2, NP, HALF), agg.reshape(2, NP, HALF),
                       w1s[2], b1s[2], w2s[2], b2s[2], batp)


# speedup vs baseline: 10.9103x; 1.0524x over previous
import functools

import jax
import jax.numpy as jnp
from jax import lax
from jax.experimental import pallas as pl
from jax.experimental.pallas import tpu as pltpu
from jax.experimental.pallas import tpu_sc as plsc

N = 50000
E = 800000
D = 64
G = 128
HALF = 32
NSUB = 16

NP = 50176
NROWS_T = NP // NSUB
ZROWS = 98

CE = 128
KI = 8
EP = 16 * 50176
NCHT = 392
NBLK_E = NCHT // KI

ATAB = 1152
NA = 14
CA = 9 * NA
NCHA = NROWS_T // NA
NBLK_A = NCHA // KI
CTAB = 512

BM = 1024
NB = NP // BM


@functools.cache
def _mesh():
    return plsc.VectorSubcoreMesh(core_axis_name="c", subcore_axis_name="s",
                                  num_cores=2, num_subcores=NSUB)


def _zero_acc(zbuf_v, acc, s):
    zero16 = jnp.zeros((16,), jnp.float32)

    def zb(i, carry):
        zbuf_v[i, pl.ds(0, 16)] = zero16
        zbuf_v[i, pl.ds(16, 16)] = zero16
        return carry

    lax.fori_loop(0, ZROWS, zb, 0)

    def zc(k, carry):
        pltpu.sync_copy(zbuf_v, acc.at[pl.ds(s * NROWS_T + k * ZROWS, ZROWS)])
        return carry

    lax.fori_loop(0, NROWS_T // ZROWS, zc, 0)


def _sc_atom_body(xidx_hbm, atab_hbm, out_hbm,
                  idx_v, rows_v, nacc_v, atab_spm,
                  semi, semg0, semg1, semw0, semw1):
    c = lax.axis_index("c")
    s = lax.axis_index("s")
    pltpu.sync_copy(atab_hbm.at[pl.ds(c * ATAB + s * (ATAB // NSUB),
                                      ATAB // NSUB)],
                    atab_spm.at[pl.ds(s * (ATAB // NSUB), ATAB // NSUB)])
    plsc.subcore_barrier()

    ibase = s * NCHA
    semw = (semw0, semw1)
    semg = (semg0, semg1)

    def do_block(b, q, first):
        pltpu.async_copy(xidx_hbm.at[pl.ds(ibase + b * KI, KI)],
                         idx_v, semi).wait()
        pg = pltpu.async_copy(atab_spm.at[idx_v.at[0]], rows_v.at[0], semg[0])
        if not first:
            pltpu.make_async_copy(out_hbm.at[pl.ds(0, KI * NA)],
                                  nacc_v.at[q], semw[q]).wait()
        for j in range(KI):
            p = j % 2
            if j < KI - 1:
                ng = pltpu.async_copy(atab_spm.at[idx_v.at[j + 1]],
                                      rows_v.at[1 - p], semg[1 - p])
            pg.wait()
            if j < KI - 1:
                pg = ng
            rv = rows_v.at[p]

            def node(m, ncarry):
                t = m * 9
                for lo in (0, 16):
                    r0 = rv[t, pl.ds(lo, 16)] + rv[t + 1, pl.ds(lo, 16)]
                    r1 = rv[t + 2, pl.ds(lo, 16)] + rv[t + 3, pl.ds(lo, 16)]
                    r2 = rv[t + 4, pl.ds(lo, 16)] + rv[t + 5, pl.ds(lo, 16)]
                    r3 = rv[t + 6, pl.ds(lo, 16)] + rv[t + 7, pl.ds(lo, 16)]
                    r4 = (r0 + r1) + (r2 + r3)
                    nacc_v[q, j * NA + m, pl.ds(lo, 16)] = (
                        r4 + rv[t + 8, pl.ds(lo, 16)])
                return ncarry

            lax.fori_loop(0, NA, node, 0)
        pltpu.async_copy(
            nacc_v.at[q],
            out_hbm.at[pl.ds(c * NP + s * NROWS_T + b * (KI * NA), KI * NA)],
            semw[q])
        return 0

    do_block(0, 0, True)
    do_block(1, 1, True)

    def pair(bp, carry):
        do_block(2 * bp, 0, False)
        do_block(2 * bp + 1, 1, False)
        return carry

    lax.fori_loop(1, NBLK_A // 2, pair, 0)
    pltpu.make_async_copy(out_hbm.at[pl.ds(0, KI * NA)],
                          nacc_v.at[0], semw0).wait()
    pltpu.make_async_copy(out_hbm.at[pl.ds(0, KI * NA)],
                          nacc_v.at[1], semw1).wait()


def _sc_layer_body(srcS_hbm, dstS_hbm, codeS_hbm, h_hbm, comb_hbm, out_hbm,
                   sidx_v, didx_v, cidx_v, hrow_v, erow_v, zbuf_v,
                   comb_spm, acc, semi, semh0, semh1, seme0, seme1,
                   sems0, sems1):
    c = lax.axis_index("c")
    s = lax.axis_index("s")
    pltpu.sync_copy(comb_hbm.at[pl.ds(c * CTAB + s * (CTAB // NSUB),
                                      CTAB // NSUB)],
                    comb_spm.at[pl.ds(s * (CTAB // NSUB), CTAB // NSUB)])
    _zero_acc(zbuf_v, acc, s)
    plsc.subcore_barrier()

    sbase = c * (EP // CE) + s * NCHT
    obase = s * NCHT
    sems = (sems0, sems1)

    def wait_scatter(q):
        pltpu.make_async_copy(h_hbm.at[pl.ds(0, CE)], hrow_v.at[q],
                              sems[q]).wait()

    def do_block(b, first):
        r1 = pltpu.async_copy(srcS_hbm.at[pl.ds(sbase + b * KI, KI)],
                              sidx_v, semi)
        r2 = pltpu.async_copy(codeS_hbm.at[pl.ds(obase + b * KI, KI)],
                              cidx_v, semi)
        r3 = pltpu.async_copy(dstS_hbm.at[pl.ds(obase + b * KI, KI)],
                              didx_v, semi)
        r1.wait()
        r2.wait()
        r3.wait()
        if not first:
            wait_scatter(0)
        ph = pltpu.async_copy(h_hbm.at[sidx_v.at[0]], hrow_v.at[0], semh0)
        pe = pltpu.async_copy(comb_spm.at[cidx_v.at[0]], erow_v.at[0], seme0)
        for j in range(KI):
            p = j % 2
            if j < KI - 1:
                if not first or j >= 1:
                    wait_scatter(1 - p)
                nh = pltpu.async_copy(h_hbm.at[sidx_v.at[j + 1]],
                                      hrow_v.at[1 - p],
                                      semh1 if p == 0 else semh0)
                ne = pltpu.async_copy(comb_spm.at[cidx_v.at[j + 1]],
                                      erow_v.at[1 - p],
                                      seme1 if p == 0 else seme0)
            ph.wait()
            pe.wait()
            if j < KI - 1:
                ph = nh
                pe = ne
            hr = hrow_v.at[p]
            er = erow_v.at[p]

            def vec(r, vcarry):
                i0 = r * 8
                for dd in range(8):
                    for lo in (0, 16):
                        a = (hr[i0 + dd, pl.ds(lo, 16)]
                             + er[i0 + dd, pl.ds(lo, 16)])
                        hr[i0 + dd, pl.ds(lo, 16)] = jnp.maximum(a, 0.0)
                return vcarry

            lax.fori_loop(0, CE // 8, vec, 0)
            pltpu.async_copy(hr, acc.at[didx_v.at[j]], sems[p], add=True)
        return 0

    do_block(0, True)

    def block(b, carry):
        return do_block(b, False)

    lax.fori_loop(1, NBLK_E, block, 0)
    wait_scatter(0)
    wait_scatter(1)
    plsc.subcore_barrier()
    pltpu.sync_copy(acc.at[pl.ds(s * NROWS_T, NROWS_T)],
                    out_hbm.at[pl.ds(c * NP + s * NROWS_T, NROWS_T)])


@functools.cache
def _sc_atom():
    return pl.kernel(
        _sc_atom_body,
        out_type=jax.ShapeDtypeStruct((2 * NP, HALF), jnp.float32),
        mesh=_mesh(),
        compiler_params=pltpu.CompilerParams(use_tc_tiling_on_sc=False),
        scratch_types=[
            pltpu.VMEM((KI, CA), jnp.int32),
            pltpu.VMEM((2, CA, HALF), jnp.float32),
            pltpu.VMEM((2, KI * NA, HALF), jnp.float32),
            pltpu.VMEM_SHARED((ATAB, HALF), jnp.float32),
            pltpu.SemaphoreType.DMA,
            pltpu.SemaphoreType.DMA,
            pltpu.SemaphoreType.DMA,
            pltpu.SemaphoreType.DMA,
            pltpu.SemaphoreType.DMA,
        ],
    )


@functools.cache
def _sc_layer():
    return pl.kernel(
        _sc_layer_body,
        out_type=jax.ShapeDtypeStruct((2 * NP, HALF), jnp.float32),
        mesh=_mesh(),
        compiler_params=pltpu.CompilerParams(use_tc_tiling_on_sc=False),
        scratch_types=[
            pltpu.VMEM((KI, CE), jnp.int32),
            pltpu.VMEM((KI, CE), jnp.int32),
            pltpu.VMEM((KI, CE), jnp.int32),
            pltpu.VMEM((2, CE, HALF), jnp.float32),
            pltpu.VMEM((2, CE, HALF), jnp.float32),
            pltpu.VMEM((ZROWS, HALF), jnp.float32),
            pltpu.VMEM_SHARED((CTAB, HALF), jnp.float32),
            pltpu.VMEM_SHARED((NP, HALF), jnp.float32),
            pltpu.SemaphoreType.DMA,
            pltpu.SemaphoreType.DMA,
            pltpu.SemaphoreType.DMA,
            pltpu.SemaphoreType.DMA,
            pltpu.SemaphoreType.DMA,
            pltpu.SemaphoreType.DMA,
            pltpu.SemaphoreType.DMA,
        ],
    )


def _mlp_compute(h3, a3, w1, b1, w2, b2, last):
    zl = h3[0] + a3[0]
    zh = h3[1] + a3[1]
    z1 = jnp.maximum(
        jnp.dot(zl, w1[:HALF], preferred_element_type=jnp.float32)
        + jnp.dot(zh, w1[HALF:], preferred_element_type=jnp.float32)
        + b1[...], 0.0)
    h2 = jnp.dot(z1, w2[...], preferred_element_type=jnp.float32) + b2[...]
    if not last:
        h2 = jnp.maximum(h2, 0.0)
    return h2


def _mlp_body(h3, a3, w1, b1, w2, b2, o3):
    h2 = _mlp_compute(h3, a3, w1, b1, w2, b2, last=False)
    o3[0] = h2[:, :HALF]
    o3[1] = h2[:, HALF:]


def _final_body(h3, a3, w1, b1, w2, b2, bat, out):
    i = pl.program_id(0)
    h2 = _mlp_compute(h3, a3, w1, b1, w2, b2, last=True)
    bids = bat[0]
    gids = lax.broadcasted_iota(jnp.int32, (G, BM), 0)
    rows = lax.broadcasted_iota(jnp.int32, (G, BM), 1) + i * BM
    onehot = jnp.where((bids == gids) & (rows < N), 1.0, 0.0)
    part = jnp.dot(onehot, h2, preferred_element_type=jnp.float32)

    @pl.when(i == 0)
    def _():
        out[...] = part

    @pl.when(i > 0)
    def _():
        out[...] = out[...] + part


_h3_spec = pl.BlockSpec((2, BM, HALF), lambda i: (0, i, 0))
_w_spec = pl.BlockSpec((D, D), lambda i: (0, 0))
_b_spec = pl.BlockSpec((1, D), lambda i: (0, 0))

_mlp_call = pl.pallas_call(
    _mlp_body,
    grid=(NB,),
    in_specs=[_h3_spec, _h3_spec, _w_spec, _b_spec, _w_spec, _b_spec],
    out_specs=pl.BlockSpec((2, BM, HALF), lambda i: (0, i, 0)),
    out_shape=jax.ShapeDtypeStruct((2, NP, HALF), jnp.float32),
)

_final_call = pl.pallas_call(
    _final_body,
    grid=(NB,),
    in_specs=[_h3_spec, _h3_spec, _w_spec, _b_spec, _w_spec, _b_spec,
              pl.BlockSpec((1, 1, BM), lambda i: (i, 0, 0))],
    out_specs=pl.BlockSpec((G, D), lambda i: (0, 0)),
    out_shape=jax.ShapeDtypeStruct((G, D), jnp.float32),
)


def kernel(x, edge_index, edge_attr, batch, atom_emb, bond_emb,
           W1_0, b1_0, W2_0, b2_0,
           W1_1, b1_1, W2_1, b2_1,
           W1_2, b1_2, W2_2, b2_2):
    atab = atom_emb.reshape(ATAB, D)
    atab_stack = jnp.concatenate([atab[:, :HALF], atab[:, HALF:]], axis=0)
    comb = (bond_emb[0][:, None, None, :] + bond_emb[1][None, :, None, :]
            + bond_emb[2][None, None, :, :]).reshape(CTAB, D)
    comb_stack = jnp.concatenate([comb[:, :HALF], comb[:, HALF:]], axis=0)

    pad_n = NP - N
    xi = x.astype(jnp.int32) + (128 * jnp.arange(9, dtype=jnp.int32))[None, :]
    xpad = (jnp.arange(pad_n * 9, dtype=jnp.int32) * 37) % ATAB
    xflat = jnp.concatenate([xi.reshape(-1), xpad]).reshape(NP * 9 // CA, CA)

    pad_e = EP - E
    ar = jnp.arange(pad_e, dtype=jnp.int32)
    src = jnp.concatenate([edge_index[0], N + ar % pad_n])
    srcS = jnp.concatenate([src, src + NP]).reshape(2 * EP // CE, CE)
    dstS = jnp.concatenate([edge_index[1], N + ar % pad_n]
                           ).reshape(EP // CE, CE)
    code = edge_attr[:, 0] * 64 + edge_attr[:, 1] * 8 + edge_attr[:, 2]
    codeS = jnp.concatenate([code, ar % CTAB]).reshape(EP // CE, CE)

    batp = jnp.pad(batch, (0, NP - N)).reshape(NB, 1, BM)
    b1s = (b1_0.reshape(1, D), b1_1.reshape(1, D), b1_2.reshape(1, D))
    b2s = (b2_0.reshape(1, D), b2_1.reshape(1, D), b2_2.reshape(1, D))
    w1s = (W1_0, W1_1, W1_2)
    w2s = (W2_0, W2_1, W2_2)

    h = _sc_atom()(xflat, atab_stack)
    for l in range(2):
        agg = _sc_layer()(srcS, dstS, codeS, h, comb_stack)
        o3 = _mlp_call(h.reshape(2, NP, HALF), agg.reshape(2, NP, HALF),
                       w1s[l], b1s[l], w2s[l], b2s[l])
        h = o3.reshape(2 * NP, HALF)
    agg = _sc_layer()(srcS, dstS, codeS, h, comb_stack)
    return _final_call(h.reshape(2, NP, HALF), agg.reshape(2, NP, HALF),
                       w1s[2], b1s[2], w2s[2], b2s[2], batp)
